# baseline (speedup 1.0000x reference)
import functools

import jax
import jax.numpy as jnp
from jax import lax
from jax.experimental import pallas as pl
from jax.experimental.pallas import tpu as pltpu
from jax.experimental.pallas import tpu_sc as plsc

D = 256
CHUNK = 128
LANES = 16
MAXWIN = 26
NB = 26
NBPAD = 48


@functools.partial(jax.jit, static_argnums=(0, 1, 2))
def _build(rows_out, rows_in, nw, h, idx32, cf):
    per = (-(-rows_out // nw) + 7) // 8 * 8

    mesh = plsc.VectorSubcoreMesh(core_axis_name="c", subcore_axis_name="s")
    nc = mesh.num_cores

    @functools.partial(
        pl.kernel,
        out_type=jax.ShapeDtypeStruct((rows_out, D), jnp.float32),
        mesh=mesh,
        scratch_types=[
            pltpu.VMEM((CHUNK, D), jnp.float32),
            pltpu.VMEM((2, CHUNK, D), jnp.float32),
            pltpu.VMEM((MAXWIN, CHUNK), jnp.int32),
            pltpu.VMEM((NBPAD,), jnp.int32),
            pltpu.SemaphoreType.DMA,
            pltpu.SemaphoreType.DMA,
            pltpu.SemaphoreType.DMA,
            pltpu.SemaphoreType.DMA,
        ],
    )
    def unpool(h_hbm, idx_hbm, cf_hbm, out_hbm,
               zeros_v, rows2_v, idx2_v, cf_v, semz, semi, semh, sems):
        w = lax.axis_index("s") * nc + lax.axis_index("c")

        cfcp = pltpu.make_async_copy(
            cf_hbm.at[pl.ds(w * NBPAD, NBPAD)], cf_v, semi
        )
        cfcp.start()

        def zbody(i, carry):
            r = i // (D // LANES)
            c = (i % (D // LANES)) * LANES
            zeros_v[r, pl.ds(c, LANES)] = jnp.zeros((LANES,), jnp.float32)
            return carry

        lax.fori_loop(0, CHUNK * (D // LANES), zbody, 0)
        cfcp.wait()

        lo = w * per
        hi = jnp.minimum(lo + per, rows_out)
        nfull = (hi - lo) // CHUNK

        def zissue(j, nz):
            v = cf_v[pl.ds(j, LANES)]
            cond = v[1] - v[0] < CHUNK

            @pl.when(cond)
            def _():
                pltpu.make_async_copy(
                    zeros_v, out_hbm.at[pl.ds(lo + j * CHUNK, CHUNK)], semz
                ).start()

            return nz + cond.astype(jnp.int32)

        nz = lax.fori_loop(0, nfull, zissue, jnp.int32(0))
        vt = cf_v[pl.ds(NB - 1, LANES)]
        e = vt[0]
        cond_t = e - vt[1] < CHUNK

        @pl.when(cond_t)
        def _():
            pltpu.make_async_copy(
                zeros_v, out_hbm.at[pl.ds(hi - CHUNK, CHUNK)], semz
            ).start()

        nz = nz + cond_t.astype(jnp.int32)

        v0 = cf_v[pl.ds(0, LANES)]
        s = v0[0]
        a0 = (s // 8) * 8
        nwin = (e - a0 + CHUNK - 1) // CHUNK

        def astart(j):
            return jnp.minimum(a0 + j * CHUNK, rows_in - CHUNK)

        def iissue(j, carry):
            pltpu.make_async_copy(
                idx_hbm.at[pl.ds(astart(j), CHUNK)], idx2_v.at[j], semi
            ).start()
            return carry

        lax.fori_loop(0, nwin, iissue, 0)

        @pl.when(nwin >= 1)
        def _():
            pltpu.make_async_copy(
                h_hbm.at[pl.ds(astart(0), CHUNK)], rows2_v.at[0], semh
            ).start()

        def zdrain(j, carry):
            pltpu.make_async_copy(
                zeros_v, out_hbm.at[pl.ds(lo, CHUNK)], semz
            ).wait()
            return carry

        lax.fori_loop(0, nz, zdrain, 0)

        def idrain(j, carry):
            pltpu.make_async_copy(
                idx_hbm.at[pl.ds(0, CHUNK)], idx2_v.at[0], semi
            ).wait()
            return carry

        lax.fori_loop(0, nwin, idrain, 0)

        def scat(j, carry):
            b = j % 2
            pltpu.make_async_copy(
                h_hbm.at[pl.ds(0, CHUNK)], rows2_v.at[0], semh
            ).wait()

            @pl.when(j >= 1)
            def _():
                pltpu.make_async_copy(
                    rows2_v.at[0], out_hbm.at[idx2_v.at[0]], sems
                ).wait()

            @pl.when(j + 1 < nwin)
            def _():
                pltpu.make_async_copy(
                    h_hbm.at[pl.ds(astart(j + 1), CHUNK)], rows2_v.at[1 - b], semh
                ).start()

            pltpu.make_async_copy(
                rows2_v.at[b], out_hbm.at[idx2_v.at[j]], sems
            ).start()
            return carry

        lax.fori_loop(0, nwin, scat, 0)

        @pl.when(nwin >= 1)
        def _():
            pltpu.make_async_copy(
                rows2_v.at[0], out_hbm.at[idx2_v.at[0]], sems
            ).wait()

    return unpool(h, idx32, cf)


def kernel(g, h, idx):
    rows_out = g.shape[0]
    rows_in = h.shape[0]
    info = plsc.get_sparse_core_info()
    nw = info.num_cores * info.num_subcores

    idx32 = idx.astype(jnp.int32)
    per = (-(-rows_out // nw) + 7) // 8 * 8

    wids = jnp.arange(nw)[:, None]
    lo_w = wids * per
    hi_w = jnp.minimum(lo_w + per, rows_out)
    bounds = jnp.minimum(lo_w + jnp.arange(NB - 1)[None, :] * CHUNK, hi_w)
    bounds = jnp.concatenate(
        [bounds, hi_w, hi_w - CHUNK, jnp.zeros((nw, NBPAD - NB - 1), jnp.int32)],
        axis=1,
    )
    cf = jnp.searchsorted(idx32, bounds.reshape(-1), method="sort").astype(
        jnp.int32
    )

    return _build(rows_out, rows_in, nw, h, idx32, cf)


# speedup vs baseline: 2.7234x; 2.7234x over previous
import functools

import jax
import jax.numpy as jnp
from jax import lax
from jax.experimental import pallas as pl
from jax.experimental.pallas import tpu as pltpu
from jax.experimental.pallas import tpu_sc as plsc

D = 256
CHUNK = 128
LANES = 16
MAXWIN = 26
NCNT = 48


@functools.partial(jax.jit, static_argnums=(0, 1, 2, 3))
def _build(rows_out, rows_in, nw, ncuts_pad, h, idx32, cuts):
    per = (-(-rows_out // nw) + 7) // 8 * 8
    tail_slot = per // CHUNK + 1

    mesh = plsc.VectorSubcoreMesh(core_axis_name="c", subcore_axis_name="s")
    nc = mesh.num_cores

    @functools.partial(
        pl.kernel,
        out_type=jax.ShapeDtypeStruct((rows_out, D), jnp.float32),
        mesh=mesh,
        scratch_types=[
            pltpu.VMEM((CHUNK, D), jnp.float32),
            pltpu.VMEM((2, CHUNK, D), jnp.float32),
            pltpu.VMEM((MAXWIN, CHUNK), jnp.int32),
            pltpu.VMEM((ncuts_pad,), jnp.int32),
            pltpu.VMEM((NCNT,), jnp.int32),
            pltpu.SemaphoreType.DMA,
            pltpu.SemaphoreType.DMA,
            pltpu.SemaphoreType.DMA,
            pltpu.SemaphoreType.DMA,
        ],
    )
    def unpool(h_hbm, idx_hbm, cuts_hbm, out_hbm, zeros_v, rows2_v, idx2_v,
               cuts_v, cnt_v, semz, semi, semh, sems):
        w = lax.axis_index("s") * nc + lax.axis_index("c")

        cfcp = pltpu.make_async_copy(cuts_hbm, cuts_v, semi)
        cfcp.start()

        def zbody(i, carry):
            r = i // (D // LANES)
            c = (i % (D // LANES)) * LANES
            zeros_v[r, pl.ds(c, LANES)] = jnp.zeros((LANES,), jnp.float32)
            return carry

        lax.fori_loop(0, CHUNK * (D // LANES), zbody, 0)
        for k in range(NCNT // LANES):
            cnt_v[pl.ds(k * LANES, LANES)] = jnp.zeros((LANES,), jnp.int32)

        cfcp.wait()
        cv = cuts_v[pl.ds(w, LANES)]
        s = cv[0]
        e = cv[1]

        lo = w * per
        hi = jnp.minimum(lo + per, rows_out)
        nfull = (hi - lo) // CHUNK

        a0 = (s // 8) * 8
        nwin = (e - a0 + CHUNK - 1) // CHUNK

        def astart(j):
            return jnp.minimum(a0 + j * CHUNK, rows_in - CHUNK)

        def iissue(j, carry):
            pltpu.make_async_copy(
                idx_hbm.at[pl.ds(astart(j), CHUNK)], idx2_v.at[j], semi
            ).start()
            return carry

        lax.fori_loop(0, nwin, iissue, 0)

        @pl.when(nwin >= 1)
        def _():
            pltpu.make_async_copy(
                h_hbm.at[pl.ds(astart(0), CHUNK)], rows2_v.at[0], semh
            ).start()

        def idrain(j, carry):
            pltpu.make_async_copy(
                idx_hbm.at[pl.ds(0, CHUNK)], idx2_v.at[0], semi
            ).wait()
            return carry

        lax.fori_loop(0, nwin, idrain, 0)

        def zissue(j, nz):
            pltpu.make_async_copy(
                zeros_v, out_hbm.at[pl.ds(lo + j * CHUNK, CHUNK)], semz
            ).start()
            return nz + 1

        nz = lax.fori_loop(0, nfull, zissue, jnp.int32(0))
        pltpu.make_async_copy(
            zeros_v, out_hbm.at[pl.ds(hi - CHUNK, CHUNK)], semz
        ).start()
        nz = nz + 1

        def zdrain(j, carry):
            pltpu.make_async_copy(
                zeros_v, out_hbm.at[pl.ds(lo, CHUNK)], semz
            ).wait()
            return carry

        lax.fori_loop(0, nz, zdrain, 0)

        def scat(j, carry):
            b = j % 2
            pltpu.make_async_copy(
                h_hbm.at[pl.ds(0, CHUNK)], rows2_v.at[0], semh
            ).wait()

            @pl.when(j >= 1)
            def _():
                pltpu.make_async_copy(
                    rows2_v.at[0], out_hbm.at[idx2_v.at[0]], sems
                ).wait()

            @pl.when(j + 1 < nwin)
            def _():
                pltpu.make_async_copy(
                    h_hbm.at[pl.ds(astart(j + 1), CHUNK)], rows2_v.at[1 - b], semh
                ).start()

            pltpu.make_async_copy(
                rows2_v.at[b], out_hbm.at[idx2_v.at[j]], sems
            ).start()
            return carry

        lax.fori_loop(0, nwin, scat, 0)

        @pl.when(nwin >= 1)
        def _():
            pltpu.make_async_copy(
                rows2_v.at[0], out_hbm.at[idx2_v.at[0]], sems
            ).wait()

    return unpool(h, idx32, cuts)


def kernel(g, h, idx):
    rows_out = g.shape[0]
    rows_in = h.shape[0]
    info = plsc.get_sparse_core_info()
    nw = info.num_cores * info.num_subcores

    idx32 = idx.astype(jnp.int32)
    per = (-(-rows_out // nw) + 7) // 8 * 8
    bounds = jnp.minimum(jnp.arange(nw + 1) * per, rows_out)
    cuts = jnp.searchsorted(idx32, bounds).astype(jnp.int32)
    ncuts_pad = (-(-(nw + 1) // LANES)) * LANES
    cuts = jnp.pad(cuts, (0, ncuts_pad - (nw + 1)))

    return _build(rows_out, rows_in, nw, ncuts_pad, h, idx32, cuts)


# speedup vs baseline: 3.2447x; 1.1914x over previous
import functools

import jax
import jax.numpy as jnp
from jax import lax
from jax.experimental import pallas as pl
from jax.experimental.pallas import tpu as pltpu
from jax.experimental.pallas import tpu_sc as plsc

D = 256
CHUNK = 128
LANES = 16
MAXWIN = 26
NCNT = 48


@functools.partial(jax.jit, static_argnums=(0, 1, 2, 3))
def _build(rows_out, rows_in, nw, ncuts_pad, h, idx32, cuts):
    per = (-(-rows_out // nw) + 7) // 8 * 8
    tail_slot = per // CHUNK + 1

    mesh = plsc.VectorSubcoreMesh(core_axis_name="c", subcore_axis_name="s")
    nc = mesh.num_cores

    @functools.partial(
        pl.kernel,
        out_type=jax.ShapeDtypeStruct((rows_out, D), jnp.float32),
        mesh=mesh,
        scratch_types=[
            pltpu.VMEM((CHUNK, D), jnp.float32),
            pltpu.VMEM((2, CHUNK, D), jnp.float32),
            pltpu.VMEM((MAXWIN, CHUNK), jnp.int32),
            pltpu.VMEM((MAXWIN * CHUNK,), jnp.int32),
            pltpu.VMEM((ncuts_pad,), jnp.int32),
            pltpu.SemaphoreType.DMA,
            pltpu.SemaphoreType.DMA,
            pltpu.SemaphoreType.DMA,
            pltpu.SemaphoreType.DMA,
        ],
    )
    def unpool(h_hbm, idx_hbm, cuts_hbm, out_hbm, zeros_v, rows2_v, idx2_v,
               idxf_v, cuts_v, semz, semi, semh, sems):
        w = lax.axis_index("s") * nc + lax.axis_index("c")

        cfcp = pltpu.make_async_copy(cuts_hbm, cuts_v, semi)
        cfcp.start()

        def zbody(i, carry):
            r = i // (D // LANES)
            c = (i % (D // LANES)) * LANES
            zeros_v[r, pl.ds(c, LANES)] = jnp.zeros((LANES,), jnp.float32)
            return carry

        lax.fori_loop(0, CHUNK * (D // LANES), zbody, 0)

        cfcp.wait()
        cv = cuts_v[pl.ds(w, LANES)]
        s = cv[0]
        e = cv[1]

        lo = w * per
        hi = jnp.minimum(lo + per, rows_out)
        nfull = (hi - lo) // CHUNK

        a0 = (s // 8) * 8
        nwin = (e - a0 + CHUNK - 1) // CHUNK

        def astart(j):
            return jnp.minimum(a0 + j * CHUNK, rows_in - CHUNK)

        def iissue(j, carry):
            pltpu.make_async_copy(
                idx_hbm.at[pl.ds(astart(j), CHUNK)], idx2_v.at[j], semi
            ).start()
            pltpu.make_async_copy(
                idx_hbm.at[pl.ds(astart(j), CHUNK)],
                idxf_v.at[pl.ds(j * CHUNK, CHUNK)], semi
            ).start()
            return carry

        lax.fori_loop(0, nwin, iissue, 0)

        @pl.when(nwin >= 1)
        def _():
            pltpu.make_async_copy(
                h_hbm.at[pl.ds(astart(0), CHUNK)], rows2_v.at[0], semh
            ).start()

        def idrain(j, carry):
            pltpu.make_async_copy(
                idx_hbm.at[pl.ds(0, CHUNK)], idx2_v.at[0], semi
            ).wait()
            return carry

        lax.fori_loop(0, 2 * nwin, idrain, 0)

        base = s - a0
        seglen = e - s

        def full_chunk(b):
            def bstep(i, c):
                blo, bhi = c
                mid = (blo + bhi) // 2
                v = idxf_v[pl.ds(base + mid, LANES)]
                lt = v[0] < b
                return (jnp.where(lt, mid + 1, blo), jnp.where(lt, bhi, mid))

            p, _ = lax.fori_loop(0, 12, bstep, (jnp.int32(0), seglen))
            v0 = idxf_v[pl.ds(base + p, LANES)]
            v127 = idxf_v[pl.ds(base + p + CHUNK - 1, LANES)]
            return jnp.logical_and(
                p + CHUNK <= seglen,
                jnp.logical_and(v0[0] == b, v127[0] == b + CHUNK - 1),
            )

        def zissue(j, nz):
            skip = full_chunk(lo + j * CHUNK)

            @pl.when(jnp.logical_not(skip))
            def _():
                pltpu.make_async_copy(
                    zeros_v, out_hbm.at[pl.ds(lo + j * CHUNK, CHUNK)], semz
                ).start()

            return nz + 1 - skip.astype(jnp.int32)

        nz = lax.fori_loop(0, nfull, zissue, jnp.int32(0))
        skip_t = full_chunk(hi - CHUNK)

        @pl.when(jnp.logical_not(skip_t))
        def _():
            pltpu.make_async_copy(
                zeros_v, out_hbm.at[pl.ds(hi - CHUNK, CHUNK)], semz
            ).start()

        nz = nz + 1 - skip_t.astype(jnp.int32)

        def zdrain(j, carry):
            pltpu.make_async_copy(
                zeros_v, out_hbm.at[pl.ds(lo, CHUNK)], semz
            ).wait()
            return carry

        lax.fori_loop(0, nz, zdrain, 0)

        def scat(j, carry):
            b = j % 2
            pltpu.make_async_copy(
                h_hbm.at[pl.ds(0, CHUNK)], rows2_v.at[0], semh
            ).wait()

            @pl.when(j >= 1)
            def _():
                pltpu.make_async_copy(
                    rows2_v.at[0], out_hbm.at[idx2_v.at[0]], sems
                ).wait()

            @pl.when(j + 1 < nwin)
            def _():
                pltpu.make_async_copy(
                    h_hbm.at[pl.ds(astart(j + 1), CHUNK)], rows2_v.at[1 - b], semh
                ).start()

            pltpu.make_async_copy(
                rows2_v.at[b], out_hbm.at[idx2_v.at[j]], sems
            ).start()
            return carry

        lax.fori_loop(0, nwin, scat, 0)

        @pl.when(nwin >= 1)
        def _():
            pltpu.make_async_copy(
                rows2_v.at[0], out_hbm.at[idx2_v.at[0]], sems
            ).wait()

    return unpool(h, idx32, cuts)


def kernel(g, h, idx):
    rows_out = g.shape[0]
    rows_in = h.shape[0]
    info = plsc.get_sparse_core_info()
    nw = info.num_cores * info.num_subcores

    idx32 = idx.astype(jnp.int32)
    per = (-(-rows_out // nw) + 7) // 8 * 8
    bounds = jnp.minimum(jnp.arange(nw + 1) * per, rows_out)
    cuts = jnp.searchsorted(idx32, bounds).astype(jnp.int32)
    ncuts_pad = (-(-(nw + 1) // LANES)) * LANES
    cuts = jnp.pad(cuts, (0, ncuts_pad - (nw + 1)))

    return _build(rows_out, rows_in, nw, ncuts_pad, h, idx32, cuts)
